# Initial kernel scaffold; baseline (speedup 1.0000x reference)
import jax
import jax.numpy as jnp
from jax.experimental import pallas as pl


def kernel(feature_maps, W1, b1, W2, b2, W3, b3):
    raise NotImplementedError("write your pallas kernel here")


# speedup vs baseline: 1.0693x; 1.0693x over previous
import functools

import jax
import jax.numpy as jnp
import numpy as np
from jax import lax
from jax.experimental import pallas as pl
from jax.experimental.pallas import tpu as pltpu

_IMG_H, _IMG_W = 512, 512
_FH, _FW = 32, 32
_C_IN, _HID = 384, 512
_SIZES = (32.0, 64.0, 128.0)
_RATIOS = (0.5, 1.0, 2.0)
_A = len(_SIZES) * len(_RATIOS)
_L = 8
_PRE_NMS, _POST_NMS = 512, 128
_MIN_SCORE, _IOU_THR, _MIN_SIZE = 0.5, 0.7, 1e-3
_P = _FH * _FW
_NB = _P * _A


def _anchor_boxes():
    sy, sx = _IMG_H / _FH, _IMG_W / _FW
    cy = (jnp.arange(_FH, dtype=jnp.float32) + 0.5) * sy
    cx = (jnp.arange(_FW, dtype=jnp.float32) + 0.5) * sx
    wh = jnp.array([[s / np.sqrt(r), s * np.sqrt(r)] for s in _SIZES for r in _RATIOS],
                   dtype=jnp.float32)
    cyg, cxg = jnp.meshgrid(cy, cx, indexing='ij')
    ctr = jnp.stack([cxg, cyg], axis=-1)
    ctr = jnp.broadcast_to(ctr[:, :, None, :], (_FH, _FW, _A, 2))
    whb = jnp.broadcast_to(wh[None, None, :, :], (_FH, _FW, _A, 2))
    return jnp.concatenate([ctr, whb], axis=-1).reshape(-1, 4)


def _conv_head_body(x_ref, w1_ref, b1_ref, wh_ref, bh_ref, y_ref):
    x = x_ref[0]
    pos = lax.broadcasted_iota(jnp.int32, (1, _P), 1)
    py = pos // _FW
    px = pos % _FW
    acc = jnp.zeros((_HID, _P), jnp.float32)
    for t in range(9):
        ky, kx = t // 3 - 1, t % 3 - 1
        s = ky * _FW + kx
        ok = (py + ky >= 0) & (py + ky < _FH) & (px + kx >= 0) & (px + kx < _FW)
        xs = x if s == 0 else jnp.roll(x, -s, axis=1)
        xs = jnp.where(ok, xs, 0.0)
        acc = acc + lax.dot(w1_ref[t], xs, preferred_element_type=jnp.float32)
    h = jnp.maximum(acc + b1_ref[...], 0.0)
    y_ref[0] = lax.dot(wh_ref[...], h, preferred_element_type=jnp.float32) + bh_ref[...]


def _conv_heads(x, W1, b1, W2, b2, W3, b3):
    w1t = jnp.transpose(W1, (2, 3, 0, 1)).reshape(9, _HID, _C_IN)
    w2r = W2.reshape(_A, _HID)
    w3r = W3.reshape(_A, 4, _HID).transpose(1, 0, 2).reshape(4 * _A, _HID)
    whead = jnp.concatenate([w2r, w3r, jnp.zeros((3, _HID), jnp.float32)], axis=0)
    bhead = jnp.concatenate([b2, b3.reshape(_A, 4).T.reshape(-1),
                             jnp.zeros((3,), jnp.float32)])[:, None]
    return pl.pallas_call(
        _conv_head_body,
        grid=(_L,),
        in_specs=[
            pl.BlockSpec((1, _C_IN, _P), lambda i: (i, 0, 0)),
            pl.BlockSpec((9, _HID, _C_IN), lambda i: (0, 0, 0)),
            pl.BlockSpec((_HID, 1), lambda i: (0, 0)),
            pl.BlockSpec((48, _HID), lambda i: (0, 0)),
            pl.BlockSpec((48, 1), lambda i: (0, 0)),
        ],
        out_specs=pl.BlockSpec((1, 48, _P), lambda i: (i, 0, 0)),
        out_shape=jax.ShapeDtypeStruct((_L, 48, _P), jnp.float32),
    )(x, w1t, b1[:, None], whead, bhead)


def _iou_all(bx):
    x1, y1, x2, y2 = bx[:, 0], bx[:, 1], bx[:, 2], bx[:, 3]
    area = jnp.maximum(x2 - x1, 0.0) * jnp.maximum(y2 - y1, 0.0)
    ix1 = jnp.maximum(x1[:, None], x1[None, :])
    iy1 = jnp.maximum(y1[:, None], y1[None, :])
    ix2 = jnp.minimum(x2[:, None], x2[None, :])
    iy2 = jnp.minimum(y2[:, None], y2[None, :])
    inter = jnp.maximum(ix2 - ix1, 0.0) * jnp.maximum(iy2 - iy1, 0.0)
    return inter / (area[:, None] + area[None, :] - inter + 1e-9)


def _select_one(boxes, scores):
    sc, idx = lax.top_k(scores, _PRE_NMS)
    bx = boxes[idx]
    x1 = jnp.clip(bx[:, 0], 0.0, float(_IMG_W))
    y1 = jnp.clip(bx[:, 1], 0.0, float(_IMG_H))
    x2 = jnp.clip(bx[:, 2], 0.0, float(_IMG_W))
    y2 = jnp.clip(bx[:, 3], 0.0, float(_IMG_H))
    bx = jnp.stack([x1, y1, x2, y2], axis=-1)
    valid = ((x2 - x1) >= _MIN_SIZE) & ((y2 - y1) >= _MIN_SIZE) & (sc >= _MIN_SCORE)
    iou = _iou_all(lax.stop_gradient(bx))
    idxs = jnp.arange(_PRE_NMS)

    def body(i, active):
        sup = (iou[i] > _IOU_THR) & (idxs > i) & active[i] & active
        return active & (~sup)

    active = lax.fori_loop(0, _PRE_NMS, body, valid)
    order = jnp.argsort(jnp.where(active, 0, 1))
    sel = order[:_POST_NMS]
    m = active[sel]
    return bx[sel] * m[:, None].astype(bx.dtype), sc[sel] * m.astype(sc.dtype)


def kernel(feature_maps, W1, b1, W2, b2, W3, b3):
    anchors = _anchor_boxes()
    x = feature_maps.reshape(_L, _C_IN, _P)
    y = _conv_heads(x, W1, b1, W2, b2, W3, b3)
    logits = y[:, 0:9, :].transpose(0, 2, 1).reshape(_L, _NB)
    tx = y[:, 9:18, :].transpose(0, 2, 1).reshape(_L, _NB)
    ty = y[:, 18:27, :].transpose(0, 2, 1).reshape(_L, _NB)
    tw = y[:, 27:36, :].transpose(0, 2, 1).reshape(_L, _NB)
    th = y[:, 36:45, :].transpose(0, 2, 1).reshape(_L, _NB)
    cls = jax.nn.sigmoid(logits)
    acx, acy, aw, ah = anchors[:, 0], anchors[:, 1], anchors[:, 2], anchors[:, 3]
    cx = acx[None] + tx * aw[None]
    cy = acy[None] + ty * ah[None]
    w = aw[None] * jnp.exp(jnp.clip(tw, -10.0, 10.0))
    hh = ah[None] * jnp.exp(jnp.clip(th, -10.0, 10.0))
    boxes = jnp.stack([cx - w / 2, cy - hh / 2, cx + w / 2, cy + hh / 2], axis=-1)
    best_boxes, best_scores = jax.vmap(_select_one)(boxes, cls)
    return best_boxes, best_scores


# speedup vs baseline: 5.1701x; 4.8352x over previous
import functools

import jax
import jax.numpy as jnp
import numpy as np
from jax import lax
from jax.experimental import pallas as pl
from jax.experimental.pallas import tpu as pltpu

_IMG_H, _IMG_W = 512, 512
_FH, _FW = 32, 32
_C_IN, _HID = 384, 512
_SIZES = (32.0, 64.0, 128.0)
_RATIOS = (0.5, 1.0, 2.0)
_A = len(_SIZES) * len(_RATIOS)
_L = 8
_PRE_NMS, _POST_NMS = 512, 128
_MIN_SCORE, _IOU_THR, _MIN_SIZE = 0.5, 0.7, 1e-3
_P = _FH * _FW
_NB = _P * _A


def _anchor_boxes():
    sy, sx = _IMG_H / _FH, _IMG_W / _FW
    cy = (jnp.arange(_FH, dtype=jnp.float32) + 0.5) * sy
    cx = (jnp.arange(_FW, dtype=jnp.float32) + 0.5) * sx
    wh = jnp.array([[s / np.sqrt(r), s * np.sqrt(r)] for s in _SIZES for r in _RATIOS],
                   dtype=jnp.float32)
    cyg, cxg = jnp.meshgrid(cy, cx, indexing='ij')
    ctr = jnp.stack([cxg, cyg], axis=-1)
    ctr = jnp.broadcast_to(ctr[:, :, None, :], (_FH, _FW, _A, 2))
    whb = jnp.broadcast_to(wh[None, None, :, :], (_FH, _FW, _A, 2))
    return jnp.concatenate([ctr, whb], axis=-1).reshape(-1, 4)


def _conv_head_body(x_ref, w1_ref, b1_ref, wh_ref, bh_ref, y_ref):
    x = x_ref[0]
    pos = lax.broadcasted_iota(jnp.int32, (1, _P), 1)
    py = pos // _FW
    px = pos % _FW
    acc = jnp.zeros((_HID, _P), jnp.float32)
    for t in range(9):
        ky, kx = t // 3 - 1, t % 3 - 1
        s = ky * _FW + kx
        ok = (py + ky >= 0) & (py + ky < _FH) & (px + kx >= 0) & (px + kx < _FW)
        xs = x if s == 0 else jnp.roll(x, -s, axis=1)
        xs = jnp.where(ok, xs, 0.0)
        acc = acc + lax.dot(w1_ref[t], xs, preferred_element_type=jnp.float32)
    h = jnp.maximum(acc + b1_ref[...], 0.0)
    y_ref[0] = lax.dot(wh_ref[...], h, preferred_element_type=jnp.float32) + bh_ref[...]


def _conv_heads(x, W1, b1, W2, b2, W3, b3):
    w1t = jnp.transpose(W1, (2, 3, 0, 1)).reshape(9, _HID, _C_IN)
    w2r = W2.reshape(_A, _HID)
    w3r = W3.reshape(_A, 4, _HID).transpose(1, 0, 2).reshape(4 * _A, _HID)
    whead = jnp.concatenate([w2r, w3r, jnp.zeros((3, _HID), jnp.float32)], axis=0)
    bhead = jnp.concatenate([b2, b3.reshape(_A, 4).T.reshape(-1),
                             jnp.zeros((3,), jnp.float32)])[:, None]
    return pl.pallas_call(
        _conv_head_body,
        grid=(_L,),
        in_specs=[
            pl.BlockSpec((1, _C_IN, _P), lambda i: (i, 0, 0)),
            pl.BlockSpec((9, _HID, _C_IN), lambda i: (0, 0, 0)),
            pl.BlockSpec((_HID, 1), lambda i: (0, 0)),
            pl.BlockSpec((48, _HID), lambda i: (0, 0)),
            pl.BlockSpec((48, 1), lambda i: (0, 0)),
        ],
        out_specs=pl.BlockSpec((1, 48, _P), lambda i: (i, 0, 0)),
        out_shape=jax.ShapeDtypeStruct((_L, 48, _P), jnp.float32),
    )(x, w1t, b1[:, None], whead, bhead)


_R, _C = 72, 128


def _nms_body(sc_ref, x1_ref, y1_ref, x2_ref, y2_ref,
              ox1_ref, oy1_ref, ox2_ref, oy2_ref, osc_ref):
    sc = sc_ref[...]
    x1, y1, x2, y2 = x1_ref[...], y1_ref[...], x2_ref[...], y2_ref[...]
    u = lax.bitcast_convert_type(sc, jnp.int32)
    fi = (lax.broadcasted_iota(jnp.int32, (_L, _R, _C), 1) * _C
          + lax.broadcasted_iota(jnp.int32, (_L, _R, _C), 2))

    def tstep(j, t):
        cand = t | (jnp.int32(1) << (30 - j))
        cnt = jnp.sum(jnp.where(u >= cand, 1, 0), axis=(1, 2), keepdims=True)
        return jnp.where(cnt >= _PRE_NMS, cand, t)

    t = lax.fori_loop(0, 31, tstep, jnp.zeros((_L, 1, 1), jnp.int32))
    g = jnp.sum(jnp.where(u > t, 1, 0), axis=(1, 2), keepdims=True)
    m_needed = _PRE_NMS - g
    tie = u == t

    def istep(j, lo):
        cand = lo + (jnp.int32(1) << (13 - j))
        f = jnp.sum(jnp.where(tie & (fi < cand), 1, 0), axis=(1, 2), keepdims=True)
        return jnp.where(f < m_needed, cand, lo)

    lo = lax.fori_loop(0, 14, istep, jnp.zeros((_L, 1, 1), jnp.int32))
    member = (u > t) | (tie & (fi <= lo))
    valid = ((x2 - x1) >= _MIN_SIZE) & ((y2 - y1) >= _MIN_SIZE) & (sc >= _MIN_SCORE)
    area = jnp.maximum(x2 - x1, 0.0) * jnp.maximum(y2 - y1, 0.0)

    def gstep(k, alive):
        msc = jnp.where(alive > 0.0, sc, -1.0)
        mx = jnp.max(msc, axis=(1, 2), keepdims=True)
        has = mx > 0.0
        ispick = (alive > 0.0) & (sc == mx)
        pf = jnp.min(jnp.where(ispick, fi, jnp.int32(_NB)), axis=(1, 2), keepdims=True)
        first = ispick & (fi == pf)
        hasf = jnp.where(has, 1.0, 0.0)
        px1 = jnp.sum(jnp.where(first, x1, 0.0), axis=(1, 2), keepdims=True)
        py1 = jnp.sum(jnp.where(first, y1, 0.0), axis=(1, 2), keepdims=True)
        px2 = jnp.sum(jnp.where(first, x2, 0.0), axis=(1, 2), keepdims=True)
        py2 = jnp.sum(jnp.where(first, y2, 0.0), axis=(1, 2), keepdims=True)
        psc = jnp.sum(jnp.where(first, sc, 0.0), axis=(1, 2), keepdims=True)
        ox1_ref[pl.ds(k, 1), :] = (px1 * hasf)[:, 0, 0][None, :]
        oy1_ref[pl.ds(k, 1), :] = (py1 * hasf)[:, 0, 0][None, :]
        ox2_ref[pl.ds(k, 1), :] = (px2 * hasf)[:, 0, 0][None, :]
        oy2_ref[pl.ds(k, 1), :] = (py2 * hasf)[:, 0, 0][None, :]
        osc_ref[pl.ds(k, 1), :] = (psc * hasf)[:, 0, 0][None, :]
        ix1 = jnp.maximum(px1, x1)
        iy1 = jnp.maximum(py1, y1)
        ix2 = jnp.minimum(px2, x2)
        iy2 = jnp.minimum(py2, y2)
        inter = jnp.maximum(ix2 - ix1, 0.0) * jnp.maximum(iy2 - iy1, 0.0)
        pa = jnp.maximum(px2 - px1, 0.0) * jnp.maximum(py2 - py1, 0.0)
        iou = inter / (pa + area - inter + 1e-9)
        sup = (iou > _IOU_THR) & has
        return alive * jnp.where(sup | first, 0.0, 1.0)

    lax.fori_loop(0, _POST_NMS, gstep,
                  jnp.where(member & valid, 1.0, 0.0).astype(jnp.float32))


def _select_nms(sc, x1, y1, x2, y2):
    o = jax.ShapeDtypeStruct((_POST_NMS, _L), jnp.float32)
    return pl.pallas_call(
        _nms_body,
        out_shape=(o, o, o, o, o),
    )(sc, x1, y1, x2, y2)


def kernel(feature_maps, W1, b1, W2, b2, W3, b3):
    anchors = _anchor_boxes()
    x = feature_maps.reshape(_L, _C_IN, _P)
    y = _conv_heads(x, W1, b1, W2, b2, W3, b3)
    logits = y[:, 0:9, :].transpose(0, 2, 1).reshape(_L, _NB)
    tx = y[:, 9:18, :].transpose(0, 2, 1).reshape(_L, _NB)
    ty = y[:, 18:27, :].transpose(0, 2, 1).reshape(_L, _NB)
    tw = y[:, 27:36, :].transpose(0, 2, 1).reshape(_L, _NB)
    th = y[:, 36:45, :].transpose(0, 2, 1).reshape(_L, _NB)
    cls = jax.nn.sigmoid(logits)
    acx, acy, aw, ah = anchors[:, 0], anchors[:, 1], anchors[:, 2], anchors[:, 3]
    cx = acx[None] + tx * aw[None]
    cy = acy[None] + ty * ah[None]
    w = aw[None] * jnp.exp(jnp.clip(tw, -10.0, 10.0))
    hh = ah[None] * jnp.exp(jnp.clip(th, -10.0, 10.0))
    x1 = jnp.clip(cx - w / 2, 0.0, float(_IMG_W)).reshape(_L, _R, _C)
    y1 = jnp.clip(cy - hh / 2, 0.0, float(_IMG_H)).reshape(_L, _R, _C)
    x2 = jnp.clip(cx + w / 2, 0.0, float(_IMG_W)).reshape(_L, _R, _C)
    y2 = jnp.clip(cy + hh / 2, 0.0, float(_IMG_H)).reshape(_L, _R, _C)
    ox1, oy1, ox2, oy2, osc = _select_nms(cls.reshape(_L, _R, _C), x1, y1, x2, y2)
    best_boxes = jnp.stack([ox1.T, oy1.T, ox2.T, oy2.T], axis=-1)
    return best_boxes, osc.T
